# baseline (speedup 1.0000x reference)
import functools

import jax
import jax.numpy as jnp
from jax import lax
from jax.experimental import pallas as pl
from jax.experimental.pallas import tpu as pltpu
from jax.experimental.pallas import tpu_sc as plsc

POS_DIM = 3
NUM_GRAPHS = 16
HP = 576
INTERP = False
NSC = 2
NSUB = 16
NW = NSC * NSUB
GCB = 64
GNBUF = 4


def _sc_gather(tab, idx):
    e2 = idx.shape[0]
    n_tab, width = tab.shape
    per_w = e2 // NW
    nit = per_w // GCB
    slab = n_tab // NSUB
    mesh = plsc.VectorSubcoreMesh(core_axis_name="c", subcore_axis_name="s",
                                  num_cores=NSC, num_subcores=NSUB)

    def body(tab_hbm, idx_hbm, out_hbm, tab_sh, *rest):
        ibufs = rest[:GNBUF]
        bufs = rest[GNBUF:2 * GNBUF]
        isems = rest[2 * GNBUF:3 * GNBUF]
        gsems = rest[3 * GNBUF:4 * GNBUF]
        wsems = rest[4 * GNBUF:5 * GNBUF]
        c = lax.axis_index("c")
        s = lax.axis_index("s")
        wid = s * NSC + c
        base = wid * per_w
        pltpu.sync_copy(tab_hbm.at[pl.ds(s * slab, slab)],
                        tab_sh.at[pl.ds(s * slab, slab)])
        plsc.subcore_barrier()

        def fire_idx(j, b):
            pltpu.async_copy(idx_hbm.at[pl.ds(base + j * GCB, GCB)],
                             ibufs[b], isems[b])

        def iwait(b):
            pltpu.make_async_copy(idx_hbm.at[pl.ds(0, GCB)], ibufs[b],
                                  isems[b]).wait()

        def fire(b):
            pltpu.async_copy(tab_sh.at[ibufs[b]], bufs[b], gsems[b])

        def gwait(b):
            pltpu.make_async_copy(tab_sh.at[ibufs[b]], bufs[b],
                                  gsems[b]).wait()

        def fire_write(j, b):
            pltpu.async_copy(bufs[b], out_hbm.at[pl.ds(base + j * GCB, GCB)],
                             wsems[b])

        def wwait(b):
            pltpu.make_async_copy(bufs[b], out_hbm.at[pl.ds(0, GCB)],
                                  wsems[b]).wait()

        for b in range(GNBUF):
            fire_idx(b, b)

        @pl.loop(0, nit, step=GNBUF)
        def _(j0):
            for b in range(GNBUF):
                iwait(b)
                fire(b)
            for b in range(GNBUF):
                gwait(b)
                fire_write(j0 + b, b)
            for b in range(GNBUF):
                wwait(b)

                @pl.when(j0 + GNBUF + b < nit)
                def _():
                    fire_idx(j0 + GNBUF + b, b)

    call = pl.kernel(
        body,
        out_type=jax.ShapeDtypeStruct((e2, width), jnp.int32),
        mesh=mesh,
        scratch_types=(
            [pltpu.VMEM_SHARED((n_tab, width), jnp.int32)]
            + [pltpu.VMEM((GCB,), jnp.int32)] * GNBUF
            + [pltpu.VMEM((GCB, width), jnp.int32)] * GNBUF
            + [pltpu.SemaphoreType.DMA] * (3 * GNBUF)
        ),
    )
    return call(tab, idx)


def _sc_scatter(rows, dst, n):
    e, ow = rows.shape
    per_w = e // NW
    nit = per_w // GCB
    n_pad = (n + 127) // 128 * 128
    per_init = n_pad // NSUB
    mesh = plsc.VectorSubcoreMesh(core_axis_name="c", subcore_axis_name="s",
                                  num_cores=NSC, num_subcores=NSUB)

    def body(rows_hbm, dst_hbm, zero_hbm, out_hbm, acc_sh,
             idx0, idx1, buf0, buf1, lsem0, lsem1, ssem0, ssem1):
        c = lax.axis_index("c")
        s = lax.axis_index("s")
        idxs = (idx0, idx1)
        bufs = (buf0, buf1)
        lsems = (lsem0, lsem1)
        ssems = (ssem0, ssem1)
        base = (c * NSUB + s) * per_w
        pltpu.sync_copy(zero_hbm.at[pl.ds(s * per_init, per_init)],
                        acc_sh.at[pl.ds(s * per_init, per_init)])
        plsc.subcore_barrier()

        def prefetch(j, b):
            eb = base + j * GCB
            pltpu.async_copy(dst_hbm.at[pl.ds(eb, GCB)], idxs[b], lsems[b])
            pltpu.async_copy(rows_hbm.at[pl.ds(eb, GCB)], bufs[b], lsems[b])

        def lwait(b):
            pltpu.make_async_copy(dst_hbm.at[pl.ds(0, GCB)], idxs[b],
                                  lsems[b]).wait()
            pltpu.make_async_copy(rows_hbm.at[pl.ds(0, GCB)], bufs[b],
                                  lsems[b]).wait()

        def fire_scatter(b):
            pltpu.async_copy(bufs[b], acc_sh.at[idxs[b]], ssems[b], add=True)

        def swait(b):
            pltpu.make_async_copy(bufs[b], acc_sh.at[idxs[b]],
                                  ssems[b]).wait()

        prefetch(0, 0)
        prefetch(1, 1)

        @pl.loop(0, nit, step=2)
        def _(j0):
            for b in range(2):
                lwait(b)
                fire_scatter(b)
            for b in range(2):
                swait(b)

                @pl.when(j0 + 2 + b < nit)
                def _():
                    prefetch(j0 + 2 + b, b)

        plsc.subcore_barrier()
        pltpu.sync_copy(acc_sh.at[pl.ds(s * per_init, per_init)],
                        out_hbm.at[c].at[pl.ds(s * per_init, per_init)])

    call = pl.kernel(
        body,
        out_type=jax.ShapeDtypeStruct((NSC, n_pad, ow), jnp.float32),
        mesh=mesh,
        scratch_types=[
            pltpu.VMEM_SHARED((n_pad, ow), jnp.float32),
            pltpu.VMEM((GCB,), jnp.int32),
            pltpu.VMEM((GCB,), jnp.int32),
            pltpu.VMEM((GCB, ow), jnp.float32),
            pltpu.VMEM((GCB, ow), jnp.float32),
            pltpu.SemaphoreType.DMA,
            pltpu.SemaphoreType.DMA,
            pltpu.SemaphoreType.DMA,
            pltpu.SemaphoreType.DMA,
        ],
    )
    return call(rows, dst, jnp.zeros((n_pad, ow), jnp.float32))[:, :n]


def _silu(v):
    return v * jax.nn.sigmoid(v)


def _unpack_feats(gi):
    lo = lax.bitcast_convert_type(gi << 16, jnp.float32)
    hi = lax.bitcast_convert_type(gi & jnp.int32(-65536), jnp.float32)
    return jnp.concatenate([lo, hi], axis=1).astype(jnp.bfloat16)


def _edge_body(gj_ref, gi_ref, ea_ref,
               w1i, w2, b2, c1, c1b, c2v, c2b,
               out_ref, *, need_coor):
    gj = gj_ref[...]
    gi = gi_ref[...]
    xj = _unpack_feats(gj[:, :64])
    xi = _unpack_feats(gi[:, :64])
    pj = lax.bitcast_convert_type(gj[:, 64:72], jnp.float32)
    pi = lax.bitcast_convert_type(gi[:, 64:72], jnp.float32)
    rel = pj - pi
    rd = jnp.sum(rel * rel, axis=-1, keepdims=True)
    te = rd.shape[0]
    xfull = jnp.concatenate(
        [xi, xj, ea_ref[...], rd.astype(jnp.bfloat16),
         jnp.ones((te, 1), jnp.bfloat16),
         jnp.zeros((te, 6), jnp.bfloat16)], axis=1)
    pre = jnp.dot(xfull, w1i[...], preferred_element_type=jnp.float32)
    h = _silu(pre).astype(jnp.bfloat16)
    m = jnp.dot(h, w2[...], preferred_element_type=jnp.float32) + b2[...]
    m = _silu(m)
    if need_coor:
        cw = jnp.dot(m.astype(jnp.bfloat16), c1[...],
                     preferred_element_type=jnp.float32) + c1b[...]
        cw = _silu(cw)
        w = jnp.sum(cw * c2v[...], axis=-1, keepdims=True) + c2b[0, 0]
        out_ref[...] = jnp.concatenate(
            [m, w * rel, jnp.zeros((te, 56), jnp.float32)], axis=1)
    else:
        out_ref[...] = jnp.concatenate(
            [m, jnp.zeros((te, 64), jnp.float32)], axis=1)


def _edge_call(g, ea16, wp, need_coor, te=2560):
    e2 = g.shape[0]
    ep = e2 // 2
    nb = ep // te
    grid = (nb,)
    row = lambda i: (i, 0)
    full = lambda i: (0, 0)
    in_specs = [
        pl.BlockSpec((te, 128), row),
        pl.BlockSpec((te, 128), lambda i: (i + nb, 0)),
        pl.BlockSpec((te, 16), row),
    ] + [pl.BlockSpec(w.shape, full) for w in wp]
    return pl.pallas_call(
        functools.partial(_edge_body, need_coor=need_coor),
        grid=grid,
        in_specs=in_specs,
        out_specs=pl.BlockSpec((te, 128), row),
        out_shape=jax.ShapeDtypeStruct((ep, 128), jnp.float32),
        interpret=INTERP,
    )(g, g, ea16, *wp)


def _node_body(f_ref, acc_ref, pos_ref, n1a, n1b, nb1, n2, nb2, nw, nb,
               fout_ref, pout_ref, *, upd_pos):
    f = f_ref[...]
    mu = jnp.mean(f)
    fc = f - mu
    std = jnp.sqrt(jnp.mean(fc * fc))
    normed = fc / (std + 1e-5) * nw[...] + nb[...]
    acc = acc_ref[0] + acc_ref[1]
    mi = acc[:, :64]
    hh = (jnp.dot(normed, n1a[...], preferred_element_type=jnp.float32)
          + jnp.dot(mi, n1b[...], preferred_element_type=jnp.float32)
          + nb1[...])
    hh = _silu(hh)
    fout_ref[...] = (jnp.dot(hh, n2[...], preferred_element_type=jnp.float32)
                     + nb2[...] + f)
    if upd_pos:
        pout_ref[...] = pos_ref[...] + acc[:, 64:72]
    else:
        pout_ref[...] = pos_ref[...]


def _node_call(feats, acc, pos8, wp, upd_pos):
    n = feats.shape[0]
    return pl.pallas_call(
        functools.partial(_node_body, upd_pos=upd_pos),
        out_shape=[jax.ShapeDtypeStruct((n, 128), jnp.float32),
                   jax.ShapeDtypeStruct((n, 8), jnp.float32)],
        interpret=INTERP,
    )(feats, acc, pos8, *wp)


def _head_body(f_ref, b_ref, m1, m1b, m2, m2b, z_ref):
    f = f_ref[...]
    b = b_ref[...]
    oh = (b == lax.broadcasted_iota(jnp.int32, (1, NUM_GRAPHS), 1)
          ).astype(jnp.float32)
    sums = lax.dot_general(oh, f, (((0,), (0,)), ((), ())),
                           preferred_element_type=jnp.float32)
    cnt = lax.dot_general(oh, jnp.ones_like(b, jnp.float32),
                          (((0,), (0,)), ((), ())),
                          preferred_element_type=jnp.float32)
    gmean = sums / jnp.maximum(cnt, 1.0)
    h = jnp.maximum(
        jnp.dot(gmean, m1[...], preferred_element_type=jnp.float32) + m1b[...],
        0.0)
    z_ref[...] = jnp.dot(h, m2[...], preferred_element_type=jnp.float32) + m2b[...]


def _head_call(feats, batch2d, wp, out_dim):
    return pl.pallas_call(
        _head_body,
        out_shape=jax.ShapeDtypeStruct((NUM_GRAPHS, out_dim), jnp.float32),
        interpret=INTERP,
    )(feats, batch2d, *wp)


def _prep_edge_weights(p):
    w1 = p["edge1"]["w"]
    h = w1.shape[1]
    wf = jnp.zeros((280, HP), jnp.float32)
    wf = wf.at[:273, :h].set(w1).at[273, :h].set(p["edge1"]["b"])
    w2p = jnp.zeros((HP, 64), jnp.float32).at[:h].set(p["edge2"]["w"])
    return [
        wf.astype(jnp.bfloat16),
        w2p.astype(jnp.bfloat16),
        p["edge2"]["b"][None, :],
        p["coors1"]["w"].astype(jnp.bfloat16),
        p["coors1"]["b"][None, :],
        p["coors2"]["w"].T,
        p["coors2"]["b"][None, :],
    ]


def _prep_node_weights(p):
    n1 = p["node1"]["w"]
    return [
        n1[0:128], n1[128:192], p["node1"]["b"][None, :],
        p["node2"]["w"], p["node2"]["b"][None, :],
        p["norm_w"][None, :], p["norm_b"][None, :],
    ]


def _pack_tab(feats, pos8):
    n = feats.shape[0]
    n_tab = (n + NW * 8 - 1) // (NW * 8) * (NW * 8)
    f16 = feats.astype(jnp.bfloat16)
    pair = jnp.stack([f16[:, :64], f16[:, 64:]], axis=-1)
    fi = lax.bitcast_convert_type(pair, jnp.int32)
    pi = lax.bitcast_convert_type(pos8, jnp.int32)
    tab = jnp.zeros((n_tab, 128), jnp.int32)
    tab = tab.at[:n, :64].set(fi).at[:n, 64:72].set(pi)
    return tab


def kernel(x, pos, edge_index, edge_attr, batch, params):
    n = x.shape[0]
    e = edge_index.shape[1]
    src = edge_index[0]
    dst = edge_index[1]
    align = 20480
    ep = (e + align - 1) // align * align
    padi = jnp.zeros((ep - e,), jnp.int32)
    idx2 = jnp.concatenate([src, padi, dst, padi])
    dstp = jnp.concatenate([dst, jnp.full((ep - e,), n, jnp.int32)])
    pos8 = jnp.zeros((n, 8), jnp.float32).at[:, :POS_DIM].set(pos)
    ea16 = jnp.zeros((ep, 16), jnp.bfloat16).at[:e].set(
        edge_attr.astype(jnp.bfloat16))
    feats = x
    for pname, need_coor in (("egnn1", True), ("egnn2", False)):
        p = params[pname]
        g = _sc_gather(_pack_tab(feats, pos8), idx2)
        out = _edge_call(g, ea16, _prep_edge_weights(p), need_coor)
        acc2 = _sc_scatter(out, dstp, n)
        feats, pos8 = _node_call(feats, acc2, pos8, _prep_node_weights(p),
                                 upd_pos=need_coor)
    z = _head_call(feats, batch[:, None],
                   [params["mlp1"]["w"], params["mlp1"]["b"][None, :],
                    params["mlp2"]["w"], params["mlp2"]["b"][None, :]],
                   params["mlp2"]["w"].shape[1])
    return z


# speedup vs baseline: 1.0282x; 1.0282x over previous
import functools

import jax
import jax.numpy as jnp
from jax import lax
from jax.experimental import pallas as pl
from jax.experimental.pallas import tpu as pltpu
from jax.experimental.pallas import tpu_sc as plsc

POS_DIM = 3
NUM_GRAPHS = 16
HP = 576
INTERP = False
NSC = 2
NSUB = 16
NW = NSC * NSUB
GCB = 64
GNBUF = 4


def _sc_gather(tab, idx):
    e2 = idx.shape[0]
    n_tab, width = tab.shape
    per_w = e2 // NW
    nit = per_w // GCB
    slab = n_tab // NSUB
    mesh = plsc.VectorSubcoreMesh(core_axis_name="c", subcore_axis_name="s",
                                  num_cores=NSC, num_subcores=NSUB)

    def body(tab_hbm, idx_hbm, out_hbm, tab_sh, *rest):
        ibufs = rest[:GNBUF]
        bufs = rest[GNBUF:2 * GNBUF]
        isems = rest[2 * GNBUF:3 * GNBUF]
        gsems = rest[3 * GNBUF:4 * GNBUF]
        wsems = rest[4 * GNBUF:5 * GNBUF]
        c = lax.axis_index("c")
        s = lax.axis_index("s")
        wid = s * NSC + c
        base = wid * per_w
        pltpu.sync_copy(tab_hbm.at[pl.ds(s * slab, slab)],
                        tab_sh.at[pl.ds(s * slab, slab)])
        plsc.subcore_barrier()

        def fire_idx(j, b):
            pltpu.async_copy(idx_hbm.at[pl.ds(base + j * GCB, GCB)],
                             ibufs[b], isems[b])

        def iwait(b):
            pltpu.make_async_copy(idx_hbm.at[pl.ds(0, GCB)], ibufs[b],
                                  isems[b]).wait()

        def fire(b):
            pltpu.async_copy(tab_sh.at[ibufs[b]], bufs[b], gsems[b])

        def gwait(b):
            pltpu.make_async_copy(tab_sh.at[ibufs[b]], bufs[b],
                                  gsems[b]).wait()

        def fire_write(j, b):
            pltpu.async_copy(bufs[b], out_hbm.at[pl.ds(base + j * GCB, GCB)],
                             wsems[b])

        def wwait(b):
            pltpu.make_async_copy(bufs[b], out_hbm.at[pl.ds(0, GCB)],
                                  wsems[b]).wait()

        for b in range(GNBUF):
            fire_idx(b, b)

        @pl.loop(0, nit, step=GNBUF)
        def _(j0):
            for b in range(GNBUF):
                iwait(b)
                fire(b)
            for b in range(GNBUF):
                gwait(b)
                fire_write(j0 + b, b)
            for b in range(GNBUF):
                wwait(b)

                @pl.when(j0 + GNBUF + b < nit)
                def _():
                    fire_idx(j0 + GNBUF + b, b)

    call = pl.kernel(
        body,
        out_type=jax.ShapeDtypeStruct((e2, width), jnp.int32),
        mesh=mesh,
        scratch_types=(
            [pltpu.VMEM_SHARED((n_tab, width), jnp.int32)]
            + [pltpu.VMEM((GCB,), jnp.int32)] * GNBUF
            + [pltpu.VMEM((GCB, width), jnp.int32)] * GNBUF
            + [pltpu.SemaphoreType.DMA] * (3 * GNBUF)
        ),
    )
    return call(tab, idx)


def _sc_scatter(rows, dst, n):
    e, ow = rows.shape
    per_w = e // NW
    nit = per_w // GCB
    n_pad = (n + 127) // 128 * 128
    per_init = n_pad // NSUB
    mesh = plsc.VectorSubcoreMesh(core_axis_name="c", subcore_axis_name="s",
                                  num_cores=NSC, num_subcores=NSUB)

    def body(rows_hbm, dst_hbm, zero_hbm, out_hbm, acc_sh,
             idx0, idx1, buf0, buf1, lsem0, lsem1, ssem0, ssem1):
        c = lax.axis_index("c")
        s = lax.axis_index("s")
        idxs = (idx0, idx1)
        bufs = (buf0, buf1)
        lsems = (lsem0, lsem1)
        ssems = (ssem0, ssem1)
        base = (c * NSUB + s) * per_w
        pltpu.sync_copy(zero_hbm.at[pl.ds(s * per_init, per_init)],
                        acc_sh.at[pl.ds(s * per_init, per_init)])
        plsc.subcore_barrier()

        def prefetch(j, b):
            eb = base + j * GCB
            pltpu.async_copy(dst_hbm.at[pl.ds(eb, GCB)], idxs[b], lsems[b])
            pltpu.async_copy(rows_hbm.at[pl.ds(eb, GCB)], bufs[b], lsems[b])

        def lwait(b):
            pltpu.make_async_copy(dst_hbm.at[pl.ds(0, GCB)], idxs[b],
                                  lsems[b]).wait()
            pltpu.make_async_copy(rows_hbm.at[pl.ds(0, GCB)], bufs[b],
                                  lsems[b]).wait()

        def fire_scatter(b):
            pltpu.async_copy(bufs[b], acc_sh.at[idxs[b]], ssems[b], add=True)

        def swait(b):
            pltpu.make_async_copy(bufs[b], acc_sh.at[idxs[b]],
                                  ssems[b]).wait()

        prefetch(0, 0)
        prefetch(1, 1)

        @pl.loop(0, nit, step=2)
        def _(j0):
            for b in range(2):
                lwait(b)
                fire_scatter(b)
            for b in range(2):
                swait(b)

                @pl.when(j0 + 2 + b < nit)
                def _():
                    prefetch(j0 + 2 + b, b)

        plsc.subcore_barrier()
        pltpu.sync_copy(acc_sh.at[pl.ds(s * per_init, per_init)],
                        out_hbm.at[c].at[pl.ds(s * per_init, per_init)])

    call = pl.kernel(
        body,
        out_type=jax.ShapeDtypeStruct((NSC, n_pad, ow), jnp.float32),
        mesh=mesh,
        scratch_types=[
            pltpu.VMEM_SHARED((n_pad, ow), jnp.float32),
            pltpu.VMEM((GCB,), jnp.int32),
            pltpu.VMEM((GCB,), jnp.int32),
            pltpu.VMEM((GCB, ow), jnp.float32),
            pltpu.VMEM((GCB, ow), jnp.float32),
            pltpu.SemaphoreType.DMA,
            pltpu.SemaphoreType.DMA,
            pltpu.SemaphoreType.DMA,
            pltpu.SemaphoreType.DMA,
        ],
    )
    return call(rows, dst, jnp.zeros((n_pad, ow), jnp.float32))[:, :n]


def _silu(v):
    return v * jax.nn.sigmoid(v)


def _unpack_feats(gi):
    lo = lax.bitcast_convert_type(gi << 16, jnp.float32)
    hi = lax.bitcast_convert_type(gi & jnp.int32(-65536), jnp.float32)
    return jnp.concatenate([lo, hi], axis=1).astype(jnp.bfloat16)


def _edge_body(gj_ref, gi_ref, ea_ref,
               w1i, w1e, w2, b2, c1, c1b, c2v, c2b,
               out_ref, *, need_coor):
    gj = gj_ref[...]
    gi = gi_ref[...]
    xj = _unpack_feats(gj[:, :64])
    xi = _unpack_feats(gi[:, :64])
    pj = lax.bitcast_convert_type(gj[:, 64:72], jnp.float32)
    pi = lax.bitcast_convert_type(gi[:, 64:72], jnp.float32)
    rel = pj - pi
    rd = jnp.sum(rel * rel, axis=-1, keepdims=True)
    te = rd.shape[0]
    x2 = jnp.concatenate([xi, xj], axis=1)
    aug = jnp.concatenate(
        [ea_ref[...], rd.astype(jnp.bfloat16),
         jnp.ones((te, 1), jnp.bfloat16),
         jnp.zeros((te, 6), jnp.bfloat16)], axis=1)
    pre = (jnp.dot(x2, w1i[...], preferred_element_type=jnp.float32)
           + jnp.dot(aug, w1e[...], preferred_element_type=jnp.float32))
    h = _silu(pre).astype(jnp.bfloat16)
    m = jnp.dot(h, w2[...], preferred_element_type=jnp.float32) + b2[...]
    m = _silu(m)
    if need_coor:
        cw = jnp.dot(m.astype(jnp.bfloat16), c1[...],
                     preferred_element_type=jnp.float32) + c1b[...]
        cw = _silu(cw)
        w = jnp.sum(cw * c2v[...], axis=-1, keepdims=True) + c2b[0, 0]
        out_ref[...] = jnp.concatenate(
            [m, w * rel, jnp.zeros((te, 56), jnp.float32)], axis=1)
    else:
        out_ref[...] = jnp.concatenate(
            [m, jnp.zeros((te, 64), jnp.float32)], axis=1)


def _edge_call(g, ea16, wp, need_coor, te=2560):
    e2 = g.shape[0]
    ep = e2 // 2
    nb = ep // te
    grid = (nb,)
    row = lambda i: (i, 0)
    full = lambda i: (0, 0)
    in_specs = [
        pl.BlockSpec((te, 128), row),
        pl.BlockSpec((te, 128), lambda i: (i + nb, 0)),
        pl.BlockSpec((te, 16), row),
    ] + [pl.BlockSpec(w.shape, full) for w in wp]
    return pl.pallas_call(
        functools.partial(_edge_body, need_coor=need_coor),
        grid=grid,
        in_specs=in_specs,
        out_specs=pl.BlockSpec((te, 128), row),
        out_shape=jax.ShapeDtypeStruct((ep, 128), jnp.float32),
        interpret=INTERP,
    )(g, g, ea16, *wp)


def _node_body(f_ref, acc_ref, pos_ref, n1a, n1b, nb1, n2, nb2, nw, nb,
               fout_ref, pout_ref, *, upd_pos):
    f = f_ref[...]
    mu = jnp.mean(f)
    fc = f - mu
    std = jnp.sqrt(jnp.mean(fc * fc))
    normed = fc / (std + 1e-5) * nw[...] + nb[...]
    acc = acc_ref[0] + acc_ref[1]
    mi = acc[:, :64]
    hh = (jnp.dot(normed, n1a[...], preferred_element_type=jnp.float32)
          + jnp.dot(mi, n1b[...], preferred_element_type=jnp.float32)
          + nb1[...])
    hh = _silu(hh)
    fout_ref[...] = (jnp.dot(hh, n2[...], preferred_element_type=jnp.float32)
                     + nb2[...] + f)
    if upd_pos:
        pout_ref[...] = pos_ref[...] + acc[:, 64:72]
    else:
        pout_ref[...] = pos_ref[...]


def _node_call(feats, acc, pos8, wp, upd_pos):
    n = feats.shape[0]
    return pl.pallas_call(
        functools.partial(_node_body, upd_pos=upd_pos),
        out_shape=[jax.ShapeDtypeStruct((n, 128), jnp.float32),
                   jax.ShapeDtypeStruct((n, 8), jnp.float32)],
        interpret=INTERP,
    )(feats, acc, pos8, *wp)


def _head_body(f_ref, b_ref, m1, m1b, m2, m2b, z_ref):
    f = f_ref[...]
    b = b_ref[...]
    oh = (b == lax.broadcasted_iota(jnp.int32, (1, NUM_GRAPHS), 1)
          ).astype(jnp.float32)
    sums = lax.dot_general(oh, f, (((0,), (0,)), ((), ())),
                           preferred_element_type=jnp.float32)
    cnt = lax.dot_general(oh, jnp.ones_like(b, jnp.float32),
                          (((0,), (0,)), ((), ())),
                          preferred_element_type=jnp.float32)
    gmean = sums / jnp.maximum(cnt, 1.0)
    h = jnp.maximum(
        jnp.dot(gmean, m1[...], preferred_element_type=jnp.float32) + m1b[...],
        0.0)
    z_ref[...] = jnp.dot(h, m2[...], preferred_element_type=jnp.float32) + m2b[...]


def _head_call(feats, batch2d, wp, out_dim):
    return pl.pallas_call(
        _head_body,
        out_shape=jax.ShapeDtypeStruct((NUM_GRAPHS, out_dim), jnp.float32),
        interpret=INTERP,
    )(feats, batch2d, *wp)


def _prep_edge_weights(p):
    w1 = p["edge1"]["w"]
    h = w1.shape[1]
    wf = jnp.zeros((280, HP), jnp.float32)
    wf = wf.at[:273, :h].set(w1).at[273, :h].set(p["edge1"]["b"])
    w2p = jnp.zeros((HP, 64), jnp.float32).at[:h].set(p["edge2"]["w"])
    return [
        wf[:256].astype(jnp.bfloat16),
        wf[256:].astype(jnp.bfloat16),
        w2p.astype(jnp.bfloat16),
        p["edge2"]["b"][None, :],
        p["coors1"]["w"].astype(jnp.bfloat16),
        p["coors1"]["b"][None, :],
        p["coors2"]["w"].T,
        p["coors2"]["b"][None, :],
    ]


def _prep_node_weights(p):
    n1 = p["node1"]["w"]
    return [
        n1[0:128], n1[128:192], p["node1"]["b"][None, :],
        p["node2"]["w"], p["node2"]["b"][None, :],
        p["norm_w"][None, :], p["norm_b"][None, :],
    ]


def _pack_tab(feats, pos8):
    n = feats.shape[0]
    n_tab = (n + NW * 8 - 1) // (NW * 8) * (NW * 8)
    f16 = feats.astype(jnp.bfloat16)
    pair = jnp.stack([f16[:, :64], f16[:, 64:]], axis=-1)
    fi = lax.bitcast_convert_type(pair, jnp.int32)
    pi = lax.bitcast_convert_type(pos8, jnp.int32)
    tab = jnp.zeros((n_tab, 128), jnp.int32)
    tab = tab.at[:n, :64].set(fi).at[:n, 64:72].set(pi)
    return tab


def kernel(x, pos, edge_index, edge_attr, batch, params):
    n = x.shape[0]
    e = edge_index.shape[1]
    src = edge_index[0]
    dst = edge_index[1]
    align = 20480
    ep = (e + align - 1) // align * align
    padi = jnp.zeros((ep - e,), jnp.int32)
    idx2 = jnp.concatenate([src, padi, dst, padi])
    dstp = jnp.concatenate([dst, jnp.full((ep - e,), n, jnp.int32)])
    pos8 = jnp.zeros((n, 8), jnp.float32).at[:, :POS_DIM].set(pos)
    ea16 = jnp.zeros((ep, 16), jnp.bfloat16).at[:e].set(
        edge_attr.astype(jnp.bfloat16))
    feats = x
    for pname, need_coor in (("egnn1", True), ("egnn2", False)):
        p = params[pname]
        g = _sc_gather(_pack_tab(feats, pos8), idx2)
        out = _edge_call(g, ea16, _prep_edge_weights(p), need_coor)
        acc2 = _sc_scatter(out, dstp, n)
        feats, pos8 = _node_call(feats, acc2, pos8, _prep_node_weights(p),
                                 upd_pos=need_coor)
    z = _head_call(feats, batch[:, None],
                   [params["mlp1"]["w"], params["mlp1"]["b"][None, :],
                    params["mlp2"]["w"], params["mlp2"]["b"][None, :]],
                   params["mlp2"]["w"].shape[1])
    return z


# speedup vs baseline: 1.2372x; 1.2033x over previous
import functools

import jax
import jax.numpy as jnp
from jax import lax
from jax.experimental import pallas as pl
from jax.experimental.pallas import tpu as pltpu
from jax.experimental.pallas import tpu_sc as plsc

POS_DIM = 3
NUM_GRAPHS = 16
HP = 576
INTERP = False
NSC = 2
NSUB = 16
NW = NSC * NSUB
GCB = 64
GNBUF = 4


def _sc_gather(tab, idx):
    e2 = idx.shape[0]
    n_tab, width = tab.shape
    per_w = e2 // NW
    nit = per_w // GCB
    slab = n_tab // NSUB
    mesh = plsc.VectorSubcoreMesh(core_axis_name="c", subcore_axis_name="s",
                                  num_cores=NSC, num_subcores=NSUB)

    def body(tab_hbm, idx_hbm, out_hbm, tab_sh, *rest):
        ibufs = rest[:GNBUF]
        bufs = rest[GNBUF:2 * GNBUF]
        isems = rest[2 * GNBUF:3 * GNBUF]
        gsems = rest[3 * GNBUF:4 * GNBUF]
        wsems = rest[4 * GNBUF:5 * GNBUF]
        c = lax.axis_index("c")
        s = lax.axis_index("s")
        wid = s * NSC + c
        base = wid * per_w
        pltpu.sync_copy(tab_hbm.at[pl.ds(s * slab, slab)],
                        tab_sh.at[pl.ds(s * slab, slab)])
        plsc.subcore_barrier()

        def fire_idx(j, b):
            pltpu.async_copy(idx_hbm.at[pl.ds(base + j * GCB, GCB)],
                             ibufs[b], isems[b])

        def iwait(b):
            pltpu.make_async_copy(idx_hbm.at[pl.ds(0, GCB)], ibufs[b],
                                  isems[b]).wait()

        def fire(b):
            pltpu.async_copy(tab_sh.at[ibufs[b]], bufs[b], gsems[b])

        def gwait(b):
            pltpu.make_async_copy(tab_sh.at[ibufs[b]], bufs[b],
                                  gsems[b]).wait()

        def fire_write(j, b):
            pltpu.async_copy(bufs[b], out_hbm.at[pl.ds(base + j * GCB, GCB)],
                             wsems[b])

        def wwait(b):
            pltpu.make_async_copy(bufs[b], out_hbm.at[pl.ds(0, GCB)],
                                  wsems[b]).wait()

        for b in range(GNBUF):
            fire_idx(b, b)

        @pl.loop(0, nit, step=GNBUF)
        def _(j0):
            for b in range(GNBUF):
                iwait(b)
                fire(b)
            for b in range(GNBUF):
                gwait(b)
                fire_write(j0 + b, b)
            for b in range(GNBUF):
                wwait(b)

                @pl.when(j0 + GNBUF + b < nit)
                def _():
                    fire_idx(j0 + GNBUF + b, b)

    call = pl.kernel(
        body,
        out_type=jax.ShapeDtypeStruct((e2, width), jnp.int32),
        mesh=mesh,
        scratch_types=(
            [pltpu.VMEM_SHARED((n_tab, width), jnp.int32)]
            + [pltpu.VMEM((GCB,), jnp.int32)] * GNBUF
            + [pltpu.VMEM((GCB, width), jnp.int32)] * GNBUF
            + [pltpu.SemaphoreType.DMA] * (3 * GNBUF)
        ),
    )
    return call(tab, idx)


def _sc_scatter(rows, dst, n):
    e, ow = rows.shape
    per_w = e // NW
    nit = per_w // GCB
    n_pad = (n + 127) // 128 * 128
    per_init = n_pad // NSUB
    mesh = plsc.VectorSubcoreMesh(core_axis_name="c", subcore_axis_name="s",
                                  num_cores=NSC, num_subcores=NSUB)

    def body(rows_hbm, dst_hbm, zero_hbm, out_hbm, acc_sh,
             idx0, idx1, buf0, buf1, lsem0, lsem1, ssem0, ssem1):
        c = lax.axis_index("c")
        s = lax.axis_index("s")
        idxs = (idx0, idx1)
        bufs = (buf0, buf1)
        lsems = (lsem0, lsem1)
        ssems = (ssem0, ssem1)
        base = (c * NSUB + s) * per_w
        pltpu.sync_copy(zero_hbm.at[pl.ds(s * per_init, per_init)],
                        acc_sh.at[pl.ds(s * per_init, per_init)])
        plsc.subcore_barrier()

        def prefetch(j, b):
            eb = base + j * GCB
            pltpu.async_copy(dst_hbm.at[pl.ds(eb, GCB)], idxs[b], lsems[b])
            pltpu.async_copy(rows_hbm.at[pl.ds(eb, GCB)], bufs[b], lsems[b])

        def lwait(b):
            pltpu.make_async_copy(dst_hbm.at[pl.ds(0, GCB)], idxs[b],
                                  lsems[b]).wait()
            pltpu.make_async_copy(rows_hbm.at[pl.ds(0, GCB)], bufs[b],
                                  lsems[b]).wait()

        def fire_scatter(b):
            pltpu.async_copy(bufs[b], acc_sh.at[idxs[b]], ssems[b], add=True)

        def swait(b):
            pltpu.make_async_copy(bufs[b], acc_sh.at[idxs[b]],
                                  ssems[b]).wait()

        prefetch(0, 0)
        prefetch(1, 1)

        @pl.loop(0, nit, step=2)
        def _(j0):
            for b in range(2):
                lwait(b)
                fire_scatter(b)
            for b in range(2):
                swait(b)

                @pl.when(j0 + 2 + b < nit)
                def _():
                    prefetch(j0 + 2 + b, b)

        plsc.subcore_barrier()
        pltpu.sync_copy(acc_sh.at[pl.ds(s * per_init, per_init)],
                        out_hbm.at[c].at[pl.ds(s * per_init, per_init)])

    call = pl.kernel(
        body,
        out_type=jax.ShapeDtypeStruct((NSC, n_pad, ow), jnp.float32),
        mesh=mesh,
        scratch_types=[
            pltpu.VMEM_SHARED((n_pad, ow), jnp.float32),
            pltpu.VMEM((GCB,), jnp.int32),
            pltpu.VMEM((GCB,), jnp.int32),
            pltpu.VMEM((GCB, ow), jnp.float32),
            pltpu.VMEM((GCB, ow), jnp.float32),
            pltpu.SemaphoreType.DMA,
            pltpu.SemaphoreType.DMA,
            pltpu.SemaphoreType.DMA,
            pltpu.SemaphoreType.DMA,
        ],
    )
    return call(rows, dst, jnp.zeros((n_pad, ow), jnp.float32))[:, :n]


def _silu(v):
    return v * jax.nn.sigmoid(v)


def _unpack_feats(gi):
    lo = lax.bitcast_convert_type(gi << 16, jnp.float32)
    hi = lax.bitcast_convert_type(gi & jnp.int32(-65536), jnp.float32)
    return jnp.concatenate([lo, hi], axis=1).astype(jnp.bfloat16)


def _edge_body(gj_ref, gi_ref, ea_ref,
               w1i, w1e, w1d, b1, w2, b2, c1, c1b, c2v, c2b,
               out_ref, *, need_coor):
    gj = gj_ref[...]
    gi = gi_ref[...]
    xj = _unpack_feats(gj[:, :64])
    xi = _unpack_feats(gi[:, :64])
    pj = lax.bitcast_convert_type(gj[:, 64:72], jnp.float32)
    pi = lax.bitcast_convert_type(gi[:, 64:72], jnp.float32)
    rel = pj - pi
    rd = jnp.sum(rel * rel, axis=-1, keepdims=True)
    x2 = jnp.concatenate([xi, xj], axis=1)
    pre = jnp.dot(x2, w1i[...], preferred_element_type=jnp.float32)
    pre = pre + jnp.dot(ea_ref[...], w1e[...], preferred_element_type=jnp.float32)
    pre = pre + rd * w1d[...] + b1[...]
    h = _silu(pre).astype(jnp.bfloat16)
    m = jnp.dot(h, w2[...], preferred_element_type=jnp.float32) + b2[...]
    m = _silu(m)
    te = m.shape[0]
    if need_coor:
        cw = jnp.dot(m.astype(jnp.bfloat16), c1[...],
                     preferred_element_type=jnp.float32) + c1b[...]
        cw = _silu(cw)
        w = jnp.sum(cw * c2v[...], axis=-1, keepdims=True) + c2b[0, 0]
        out_ref[...] = jnp.concatenate(
            [m, w * rel, jnp.zeros((te, 56), jnp.float32)], axis=1)
    else:
        out_ref[...] = jnp.concatenate(
            [m, jnp.zeros((te, 64), jnp.float32)], axis=1)


def _edge_call(g, ea16, ea_off, wp, need_coor, te=2560):
    e2 = g.shape[0]
    ep = e2 // 2
    nb = ep // te
    grid = (nb,)
    row = lambda i: (i, 0)
    full = lambda i: (0, 0)
    in_specs = [
        pl.BlockSpec((te, 128), row),
        pl.BlockSpec((te, 128), lambda i: (i + nb, 0)),
        pl.BlockSpec((te, 16), lambda i: (i + ea_off, 0)),
    ] + [pl.BlockSpec(w.shape, full) for w in wp]
    return pl.pallas_call(
        functools.partial(_edge_body, need_coor=need_coor),
        grid=grid,
        in_specs=in_specs,
        out_specs=pl.BlockSpec((te, 128), row),
        out_shape=jax.ShapeDtypeStruct((ep, 128), jnp.float32),
        interpret=INTERP,
    )(g, g, ea16, *wp)


def _node_body(f_ref, acc_ref, accb_ref, pos_ref, n1a, n1b, nb1, n2, nb2,
               nw, nb, fout_ref, pout_ref, *, upd_pos):
    f = f_ref[...]
    mu = jnp.mean(f)
    fc = f - mu
    std = jnp.sqrt(jnp.mean(fc * fc))
    normed = fc / (std + 1e-5) * nw[...] + nb[...]
    acc = (acc_ref[0] + acc_ref[1]) + (accb_ref[0] + accb_ref[1])
    mi = acc[:, :64]
    hh = (jnp.dot(normed, n1a[...], preferred_element_type=jnp.float32)
          + jnp.dot(mi, n1b[...], preferred_element_type=jnp.float32)
          + nb1[...])
    hh = _silu(hh)
    fout_ref[...] = (jnp.dot(hh, n2[...], preferred_element_type=jnp.float32)
                     + nb2[...] + f)
    if upd_pos:
        pout_ref[...] = pos_ref[...] + acc[:, 64:72]
    else:
        pout_ref[...] = pos_ref[...]


def _node_call(feats, acca, accb, pos8, wp, upd_pos):
    n = feats.shape[0]
    return pl.pallas_call(
        functools.partial(_node_body, upd_pos=upd_pos),
        out_shape=[jax.ShapeDtypeStruct((n, 128), jnp.float32),
                   jax.ShapeDtypeStruct((n, 8), jnp.float32)],
        interpret=INTERP,
    )(feats, acca, accb, pos8, *wp)


def _head_body(f_ref, b_ref, m1, m1b, m2, m2b, z_ref):
    f = f_ref[...]
    b = b_ref[...]
    oh = (b == lax.broadcasted_iota(jnp.int32, (1, NUM_GRAPHS), 1)
          ).astype(jnp.float32)
    sums = lax.dot_general(oh, f, (((0,), (0,)), ((), ())),
                           preferred_element_type=jnp.float32)
    cnt = lax.dot_general(oh, jnp.ones_like(b, jnp.float32),
                          (((0,), (0,)), ((), ())),
                          preferred_element_type=jnp.float32)
    gmean = sums / jnp.maximum(cnt, 1.0)
    h = jnp.maximum(
        jnp.dot(gmean, m1[...], preferred_element_type=jnp.float32) + m1b[...],
        0.0)
    z_ref[...] = jnp.dot(h, m2[...], preferred_element_type=jnp.float32) + m2b[...]


def _head_call(feats, batch2d, wp, out_dim):
    return pl.pallas_call(
        _head_body,
        out_shape=jax.ShapeDtypeStruct((NUM_GRAPHS, out_dim), jnp.float32),
        interpret=INTERP,
    )(feats, batch2d, *wp)


def _prep_edge_weights(p):
    w1 = p["edge1"]["w"]
    h = w1.shape[1]
    w1p = jnp.zeros((w1.shape[0], HP), jnp.float32).at[:, :h].set(w1)
    b1p = jnp.zeros((1, HP), jnp.float32).at[:, :h].set(p["edge1"]["b"])
    w2p = jnp.zeros((HP, 64), jnp.float32).at[:h].set(p["edge2"]["w"])
    return [
        w1p[0:256].astype(jnp.bfloat16),
        w1p[256:272].astype(jnp.bfloat16),
        w1p[272:273],
        b1p,
        w2p.astype(jnp.bfloat16),
        p["edge2"]["b"][None, :],
        p["coors1"]["w"].astype(jnp.bfloat16),
        p["coors1"]["b"][None, :],
        p["coors2"]["w"].T,
        p["coors2"]["b"][None, :],
    ]


def _prep_node_weights(p):
    n1 = p["node1"]["w"]
    return [
        n1[0:128], n1[128:192], p["node1"]["b"][None, :],
        p["node2"]["w"], p["node2"]["b"][None, :],
        p["norm_w"][None, :], p["norm_b"][None, :],
    ]


def _pack_tab(feats, pos8):
    n = feats.shape[0]
    n_tab = (n + NW * 8 - 1) // (NW * 8) * (NW * 8)
    f16 = feats.astype(jnp.bfloat16)
    pair = jnp.stack([f16[:, :64], f16[:, 64:]], axis=-1)
    fi = lax.bitcast_convert_type(pair, jnp.int32)
    pi = lax.bitcast_convert_type(pos8, jnp.int32)
    tab = jnp.zeros((n_tab, 128), jnp.int32)
    tab = tab.at[:n, :64].set(fi).at[:n, 64:72].set(pi)
    return tab


def kernel(x, pos, edge_index, edge_attr, batch, params):
    n = x.shape[0]
    e = edge_index.shape[1]
    src = edge_index[0]
    dst = edge_index[1]
    align = 40960
    ep = (e + align - 1) // align * align
    he = ep // 2
    nhtiles = he // 2560
    padn = jnp.full((ep - e,), n, jnp.int32)
    srcp = jnp.concatenate([src, padn])
    dstp = jnp.concatenate([dst, padn])
    halves = [jnp.concatenate([srcp[h * he:(h + 1) * he],
                               dstp[h * he:(h + 1) * he]]) for h in (0, 1)]
    pos8 = jnp.zeros((n, 8), jnp.float32).at[:, :POS_DIM].set(pos)
    ea16 = jnp.zeros((ep, 16), jnp.bfloat16).at[:e].set(
        edge_attr.astype(jnp.bfloat16))
    feats = x
    for pname, need_coor in (("egnn1", True), ("egnn2", False)):
        p = params[pname]
        wp = _prep_edge_weights(p)
        tab = _pack_tab(feats, pos8)
        accs = []
        for h in (0, 1):
            g = _sc_gather(tab, halves[h])
            out = _edge_call(g, ea16, h * nhtiles, wp, need_coor)
            accs.append(_sc_scatter(out, dstp[h * he:(h + 1) * he], n))
        feats, pos8 = _node_call(feats, accs[0], accs[1], pos8,
                                 _prep_node_weights(p), upd_pos=need_coor)
    z = _head_call(feats, batch[:, None],
                   [params["mlp1"]["w"], params["mlp1"]["b"][None, :],
                    params["mlp2"]["w"], params["mlp2"]["b"][None, :]],
                   params["mlp2"]["w"].shape[1])
    return z


# speedup vs baseline: 1.3094x; 1.0584x over previous
import functools

import jax
import jax.numpy as jnp
from jax import lax
from jax.experimental import pallas as pl
from jax.experimental.pallas import tpu as pltpu
from jax.experimental.pallas import tpu_sc as plsc

POS_DIM = 3
NUM_GRAPHS = 16
HP = 576
INTERP = False
NSC = 2
NSUB = 16
NW = NSC * NSUB
GCB = 64
GNBUF = 4


def _sc_gather(tab, idx):
    e2 = idx.shape[0]
    n_tab, width = tab.shape
    per_w = e2 // NW
    nit = per_w // GCB
    slab = n_tab // NSUB
    mesh = plsc.VectorSubcoreMesh(core_axis_name="c", subcore_axis_name="s",
                                  num_cores=NSC, num_subcores=NSUB)

    def body(tab_hbm, idx_hbm, out_hbm, tab_sh, *rest):
        ibufs = rest[:GNBUF]
        bufs = rest[GNBUF:2 * GNBUF]
        isems = rest[2 * GNBUF:3 * GNBUF]
        gsems = rest[3 * GNBUF:4 * GNBUF]
        wsems = rest[4 * GNBUF:5 * GNBUF]
        c = lax.axis_index("c")
        s = lax.axis_index("s")
        wid = s * NSC + c
        base = wid * per_w
        pltpu.sync_copy(tab_hbm.at[pl.ds(s * slab, slab)],
                        tab_sh.at[pl.ds(s * slab, slab)])
        plsc.subcore_barrier()

        def fire_idx(j, b):
            pltpu.async_copy(idx_hbm.at[pl.ds(base + j * GCB, GCB)],
                             ibufs[b], isems[b])

        def iwait(b):
            pltpu.make_async_copy(idx_hbm.at[pl.ds(0, GCB)], ibufs[b],
                                  isems[b]).wait()

        def fire(b):
            pltpu.async_copy(tab_sh.at[ibufs[b]], bufs[b], gsems[b])

        def gwait(b):
            pltpu.make_async_copy(tab_sh.at[ibufs[b]], bufs[b],
                                  gsems[b]).wait()

        def fire_write(j, b):
            pltpu.async_copy(bufs[b], out_hbm.at[pl.ds(base + j * GCB, GCB)],
                             wsems[b])

        def wwait(b):
            pltpu.make_async_copy(bufs[b], out_hbm.at[pl.ds(0, GCB)],
                                  wsems[b]).wait()

        for b in range(GNBUF):
            fire_idx(b, b)

        @pl.loop(0, nit, step=GNBUF)
        def _(j0):
            for b in range(GNBUF):
                iwait(b)
                fire(b)
            for b in range(GNBUF):
                gwait(b)
                fire_write(j0 + b, b)
            for b in range(GNBUF):
                wwait(b)

                @pl.when(j0 + GNBUF + b < nit)
                def _():
                    fire_idx(j0 + GNBUF + b, b)

    call = pl.kernel(
        body,
        out_type=jax.ShapeDtypeStruct((e2, width), jnp.int32),
        mesh=mesh,
        scratch_types=(
            [pltpu.VMEM_SHARED((n_tab, width), jnp.int32)]
            + [pltpu.VMEM((GCB,), jnp.int32)] * GNBUF
            + [pltpu.VMEM((GCB, width), jnp.int32)] * GNBUF
            + [pltpu.SemaphoreType.DMA] * (3 * GNBUF)
        ),
    )
    return call(tab, idx)


def _sc_scatter(rows, dst, n):
    e, ow = rows.shape
    per_w = e // NW
    nit = per_w // GCB
    n_pad = (n + 127) // 128 * 128
    per_init = n_pad // NSUB
    mesh = plsc.VectorSubcoreMesh(core_axis_name="c", subcore_axis_name="s",
                                  num_cores=NSC, num_subcores=NSUB)

    def body(rows_hbm, dst_hbm, zero_hbm, out_hbm, acc_sh,
             idx0, idx1, buf0, buf1, lsem0, lsem1, ssem0, ssem1):
        c = lax.axis_index("c")
        s = lax.axis_index("s")
        idxs = (idx0, idx1)
        bufs = (buf0, buf1)
        lsems = (lsem0, lsem1)
        ssems = (ssem0, ssem1)
        base = (c * NSUB + s) * per_w
        pltpu.sync_copy(zero_hbm.at[pl.ds(s * per_init, per_init)],
                        acc_sh.at[pl.ds(s * per_init, per_init)])
        plsc.subcore_barrier()

        def prefetch(j, b):
            eb = base + j * GCB
            pltpu.async_copy(dst_hbm.at[pl.ds(eb, GCB)], idxs[b], lsems[b])
            pltpu.async_copy(rows_hbm.at[pl.ds(eb, GCB)], bufs[b], lsems[b])

        def lwait(b):
            pltpu.make_async_copy(dst_hbm.at[pl.ds(0, GCB)], idxs[b],
                                  lsems[b]).wait()
            pltpu.make_async_copy(rows_hbm.at[pl.ds(0, GCB)], bufs[b],
                                  lsems[b]).wait()

        def fire_scatter(b):
            pltpu.async_copy(bufs[b], acc_sh.at[idxs[b]], ssems[b], add=True)

        def swait(b):
            pltpu.make_async_copy(bufs[b], acc_sh.at[idxs[b]],
                                  ssems[b]).wait()

        prefetch(0, 0)
        prefetch(1, 1)

        @pl.loop(0, nit, step=2)
        def _(j0):
            for b in range(2):
                lwait(b)
                fire_scatter(b)
            for b in range(2):
                swait(b)

                @pl.when(j0 + 2 + b < nit)
                def _():
                    prefetch(j0 + 2 + b, b)

        plsc.subcore_barrier()
        pltpu.sync_copy(acc_sh.at[pl.ds(s * per_init, per_init)],
                        out_hbm.at[c].at[pl.ds(s * per_init, per_init)])

    call = pl.kernel(
        body,
        out_type=jax.ShapeDtypeStruct((NSC, n_pad, ow), jnp.float32),
        mesh=mesh,
        scratch_types=[
            pltpu.VMEM_SHARED((n_pad, ow), jnp.float32),
            pltpu.VMEM((GCB,), jnp.int32),
            pltpu.VMEM((GCB,), jnp.int32),
            pltpu.VMEM((GCB, ow), jnp.float32),
            pltpu.VMEM((GCB, ow), jnp.float32),
            pltpu.SemaphoreType.DMA,
            pltpu.SemaphoreType.DMA,
            pltpu.SemaphoreType.DMA,
            pltpu.SemaphoreType.DMA,
        ],
    )
    return call(rows, dst, jnp.zeros((n_pad, ow), jnp.float32))[:, :n]


def _silu(v):
    return v * jax.nn.sigmoid(v)


def _unpack_feats(gi):
    lo = lax.bitcast_convert_type(gi << 16, jnp.float32)
    hi = lax.bitcast_convert_type(gi & jnp.int32(-65536), jnp.float32)
    return jnp.concatenate([lo, hi], axis=1).astype(jnp.bfloat16)


def _edge_body(gj_ref, gi_ref, ea_ref,
               w1i, w1e, w1d, b1, w2, b2, c1, c1b, c2v, c2b,
               out_ref, *, need_coor):
    gj = gj_ref[...]
    gi = gi_ref[...]
    xj = _unpack_feats(gj[:, :64])
    xi = _unpack_feats(gi[:, :64])
    pj = lax.bitcast_convert_type(gj[:, 64:72], jnp.float32)
    pi = lax.bitcast_convert_type(gi[:, 64:72], jnp.float32)
    rel = pj - pi
    rd = jnp.sum(rel * rel, axis=-1, keepdims=True)
    x2 = jnp.concatenate([xi, xj], axis=1)
    pre = jnp.dot(x2, w1i[...], preferred_element_type=jnp.float32)
    pre = pre + jnp.dot(ea_ref[...], w1e[...], preferred_element_type=jnp.float32)
    pre = pre + rd * w1d[...] + b1[...]
    h = _silu(pre).astype(jnp.bfloat16)
    m = jnp.dot(h, w2[...], preferred_element_type=jnp.float32) + b2[...]
    m = _silu(m)
    te = m.shape[0]
    if need_coor:
        cw = jnp.dot(m.astype(jnp.bfloat16), c1[...],
                     preferred_element_type=jnp.float32) + c1b[...]
        cw = _silu(cw)
        w = jnp.sum(cw * c2v[...], axis=-1, keepdims=True) + c2b[0, 0]
        out_ref[...] = jnp.concatenate(
            [m, w * rel, jnp.zeros((te, 56), jnp.float32)], axis=1)
    else:
        out_ref[...] = jnp.concatenate(
            [m, jnp.zeros((te, 64), jnp.float32)], axis=1)


def _edge_call(g, ea16, ea_off, wp, need_coor, te=2560):
    e2 = g.shape[0]
    ep = e2 // 2
    nb = ep // te
    grid = (nb,)
    row = lambda i: (i, 0)
    full = lambda i: (0, 0)
    in_specs = [
        pl.BlockSpec((te, 128), row),
        pl.BlockSpec((te, 128), lambda i: (i + nb, 0)),
        pl.BlockSpec((te, 16), lambda i: (i + ea_off, 0)),
    ] + [pl.BlockSpec(w.shape, full) for w in wp]
    return pl.pallas_call(
        functools.partial(_edge_body, need_coor=need_coor),
        grid=grid,
        in_specs=in_specs,
        out_specs=pl.BlockSpec((te, 128), row),
        out_shape=jax.ShapeDtypeStruct((ep, 128), jnp.float32),
        interpret=INTERP,
    )(g, g, ea16, *wp)


def _stats_body(f_ref, sout_ref):
    f = f_ref[...]
    mu = jnp.mean(f)
    fc = f - mu
    std = jnp.sqrt(jnp.mean(fc * fc))
    sout_ref[...] = jnp.concatenate(
        [jnp.full((1, 128), mu, jnp.float32),
         jnp.full((1, 128), std, jnp.float32),
         jnp.zeros((6, 128), jnp.float32)], axis=0)


def _node_body(*refs, nacc, upd_pos):
    (f_ref, *accrefs), rest = refs[:1 + nacc], refs[1 + nacc:]
    pos_ref, st_ref, n1a, n1b, nb1, n2, nb2, nw, nb, fout_ref, pout_ref = rest
    f = f_ref[...]
    mu = st_ref[0, 0]
    std = st_ref[1, 0]
    fc = f - mu
    normed = fc / (std + 1e-5) * nw[...] + nb[...]
    acc = accrefs[0][0] + accrefs[0][1]
    for a in accrefs[1:]:
        acc = acc + (a[0] + a[1])
    mi = acc[:, :64]
    hh = (jnp.dot(normed, n1a[...], preferred_element_type=jnp.float32)
          + jnp.dot(mi, n1b[...], preferred_element_type=jnp.float32)
          + nb1[...])
    hh = _silu(hh)
    fout_ref[...] = (jnp.dot(hh, n2[...], preferred_element_type=jnp.float32)
                     + nb2[...] + f)
    if upd_pos:
        pout_ref[...] = pos_ref[...] + acc[:, 64:72]
    else:
        pout_ref[...] = pos_ref[...]


def _node_call(feats, accs, pos8, wp, upd_pos, bn=2000):
    n = feats.shape[0]
    stats = pl.pallas_call(
        _stats_body,
        out_shape=jax.ShapeDtypeStruct((8, 128), jnp.float32),
        interpret=INTERP,
    )(feats)
    grid = (n // bn,)
    row = lambda i: (i, 0)
    full = lambda i: (0, 0)
    in_specs = ([pl.BlockSpec((bn, 128), row)]
                + [pl.BlockSpec((2, bn, 128), lambda i: (0, i, 0))] * len(accs)
                + [pl.BlockSpec((bn, 8), row), pl.BlockSpec((8, 128), full)]
                + [pl.BlockSpec(w.shape, full) for w in wp])
    return pl.pallas_call(
        functools.partial(_node_body, nacc=len(accs), upd_pos=upd_pos),
        grid=grid,
        in_specs=in_specs,
        out_specs=[pl.BlockSpec((bn, 128), row), pl.BlockSpec((bn, 8), row)],
        out_shape=[jax.ShapeDtypeStruct((n, 128), jnp.float32),
                   jax.ShapeDtypeStruct((n, 8), jnp.float32)],
        interpret=INTERP,
    )(feats, *accs, pos8, stats, *wp)


def _head_body(f_ref, b_ref, m1, m1b, m2, m2b, z_ref):
    f = f_ref[...]
    b = b_ref[...]
    oh = (b == lax.broadcasted_iota(jnp.int32, (1, NUM_GRAPHS), 1)
          ).astype(jnp.float32)
    sums = lax.dot_general(oh, f, (((0,), (0,)), ((), ())),
                           preferred_element_type=jnp.float32)
    cnt = lax.dot_general(oh, jnp.ones_like(b, jnp.float32),
                          (((0,), (0,)), ((), ())),
                          preferred_element_type=jnp.float32)
    gmean = sums / jnp.maximum(cnt, 1.0)
    h = jnp.maximum(
        jnp.dot(gmean, m1[...], preferred_element_type=jnp.float32) + m1b[...],
        0.0)
    z_ref[...] = jnp.dot(h, m2[...], preferred_element_type=jnp.float32) + m2b[...]


def _head_call(feats, batch2d, wp, out_dim):
    return pl.pallas_call(
        _head_body,
        out_shape=jax.ShapeDtypeStruct((NUM_GRAPHS, out_dim), jnp.float32),
        interpret=INTERP,
    )(feats, batch2d, *wp)


def _prep_edge_weights(p):
    w1 = p["edge1"]["w"]
    h = w1.shape[1]
    w1p = jnp.zeros((w1.shape[0], HP), jnp.float32).at[:, :h].set(w1)
    b1p = jnp.zeros((1, HP), jnp.float32).at[:, :h].set(p["edge1"]["b"])
    w2p = jnp.zeros((HP, 64), jnp.float32).at[:h].set(p["edge2"]["w"])
    return [
        w1p[0:256].astype(jnp.bfloat16),
        w1p[256:272].astype(jnp.bfloat16),
        w1p[272:273],
        b1p,
        w2p.astype(jnp.bfloat16),
        p["edge2"]["b"][None, :],
        p["coors1"]["w"].astype(jnp.bfloat16),
        p["coors1"]["b"][None, :],
        p["coors2"]["w"].T,
        p["coors2"]["b"][None, :],
    ]


def _prep_node_weights(p):
    n1 = p["node1"]["w"]
    return [
        n1[0:128], n1[128:192], p["node1"]["b"][None, :],
        p["node2"]["w"], p["node2"]["b"][None, :],
        p["norm_w"][None, :], p["norm_b"][None, :],
    ]


def _pack_tab(feats, pos8):
    n = feats.shape[0]
    n_tab = (n + NW * 8 - 1) // (NW * 8) * (NW * 8)
    f16 = feats.astype(jnp.bfloat16)
    pair = jnp.stack([f16[:, :64], f16[:, 64:]], axis=-1)
    fi = lax.bitcast_convert_type(pair, jnp.int32)
    pi = lax.bitcast_convert_type(pos8, jnp.int32)
    tab = jnp.zeros((n_tab, 128), jnp.int32)
    tab = tab.at[:n, :64].set(fi).at[:n, 64:72].set(pi)
    return tab


def kernel(x, pos, edge_index, edge_attr, batch, params):
    n = x.shape[0]
    e = edge_index.shape[1]
    src = edge_index[0]
    dst = edge_index[1]
    nchunk = 4
    align = nchunk * 20480
    ep = (e + align - 1) // align * align
    he = ep // nchunk
    nhtiles = he // 2560
    padn = jnp.full((ep - e,), n, jnp.int32)
    srcp = jnp.concatenate([src, padn])
    dstp = jnp.concatenate([dst, padn])
    halves = [jnp.concatenate([srcp[h * he:(h + 1) * he],
                               dstp[h * he:(h + 1) * he]])
              for h in range(nchunk)]
    pos8 = jnp.zeros((n, 8), jnp.float32).at[:, :POS_DIM].set(pos)
    ea16 = jnp.zeros((ep, 16), jnp.bfloat16).at[:e].set(
        edge_attr.astype(jnp.bfloat16))
    feats = x
    for pname, need_coor in (("egnn1", True), ("egnn2", False)):
        p = params[pname]
        wp = _prep_edge_weights(p)
        tab = _pack_tab(feats, pos8)
        accs = []
        for h in range(nchunk):
            g = _sc_gather(tab, halves[h])
            out = _edge_call(g, ea16, h * nhtiles, wp, need_coor)
            accs.append(_sc_scatter(out, dstp[h * he:(h + 1) * he], n))
        feats, pos8 = _node_call(feats, accs, pos8,
                                 _prep_node_weights(p), upd_pos=need_coor)
    z = _head_call(feats, batch[:, None],
                   [params["mlp1"]["w"], params["mlp1"]["b"][None, :],
                    params["mlp2"]["w"], params["mlp2"]["b"][None, :]],
                   params["mlp2"]["w"].shape[1])
    return z
